# Initial kernel scaffold; baseline (speedup 1.0000x reference)
import jax
import jax.numpy as jnp
from jax.experimental import pallas as pl


def kernel(states, goals, conv_w1, conv_b1, conv_w2, conv_b2, dec_w1, dec_b1, dec_w2, dec_b2, dec_w3, dec_b3, dec_w4, dec_b4):
    raise NotImplementedError("write your pallas kernel here")


# speedup vs baseline: 3.0141x; 3.0141x over previous
import functools

import jax
import jax.numpy as jnp
from jax.experimental import pallas as pl

_TOPK = 12
_OBS_RADIUS = 1.0
_HIGHEST = jax.lax.Precision.HIGHEST


def _controller_body(s_ref, sfull_ref, sT_ref, g_ref, w1T_ref, b1_ref,
                     w2T_ref, b2_ref, d1T_ref, b1d_ref, d2T_ref,
                     b2d_ref, d3T_ref, b3d_ref, d4T_ref, b4d_ref, o_ref,
                     *, blk, n):
    i = pl.program_id(0)
    si = s_ref[...]
    xi = si[:, 0:1]
    yi = si[:, 1:2]
    xj = sT_ref[0:1, :]
    yj = sT_ref[1:2, :]

    dxm = xi - xj
    dym = yi - yj
    work = jnp.sqrt((dxm * dxm + 1e-4) + (dym * dym + 1e-4))

    iota = jax.lax.broadcasted_iota(jnp.int32, (blk, n), 1)
    gid = i * blk + jax.lax.broadcasted_iota(jnp.int32, (blk, 1), 0)

    sfull = sfull_ref[...]
    w1T = w1T_ref[...]
    b1 = b1_ref[...]
    w2T = w2T_ref[...]
    b2 = b2_ref[...]

    cols5 = []
    masks = []
    for k in range(_TOPK):
        m = jnp.min(work, axis=1, keepdims=True)
        idxs = jnp.min(jnp.where(work == m, iota, jnp.int32(2147483647)),
                       axis=1, keepdims=True)
        ohb = iota == idxs
        work = jnp.where(ohb, jnp.float32(1e30), work)
        ohf = ohb.astype(jnp.float32)
        gth = jnp.dot(ohf, sfull, precision=_HIGHEST,
                      preferred_element_type=jnp.float32)
        diff = si - gth
        eyef = (idxs == gid).astype(jnp.float32)
        dx = diff[:, 0:1]
        dy = diff[:, 1:2]
        dist = jnp.sqrt(dx * dx + dy * dy)
        masks.append((dist < _OBS_RADIUS).astype(jnp.float32))
        cols5.append(jnp.concatenate([diff, eyef], axis=1))

    best = None
    argf = None
    for kk in range(_TOPK):
        x5 = jnp.concatenate(
            [cols5[(c * _TOPK + kk) // 5]
             [:, (c * _TOPK + kk) % 5:(c * _TOPK + kk) % 5 + 1]
             for c in range(5)], axis=1)
        h1 = jnp.dot(x5, w1T,
                     preferred_element_type=jnp.float32) + b1
        h1 = jnp.maximum(h1, 0.0)
        h2 = jnp.dot(h1, w2T,
                     preferred_element_type=jnp.float32) + b2
        h2 = jnp.maximum(h2, 0.0)
        v = h2 * masks[kk]
        if kk == 0:
            best = v
            argf = jnp.zeros_like(v)
        else:
            upd = v > best
            argf = jnp.where(upd, jnp.float32(kk), argf)
            best = jnp.where(upd, v, best)

    gi = g_ref[...]
    s40 = xi - gi[:, 0:1]
    s41 = yi - gi[:, 1:2]
    s42 = si[:, 2:3]
    s43 = si[:, 3:4]

    xloc = jnp.concatenate([argf, s40, s41, s42, s43], axis=1)
    y1 = jnp.dot(xloc, d1T_ref[...],
                 preferred_element_type=jnp.float32) + b1d_ref[...]
    y1 = jnp.maximum(y1, 0.0)
    y2 = jnp.maximum(jnp.dot(y1, d2T_ref[...],
                             preferred_element_type=jnp.float32)
                     + b2d_ref[...], 0.0)
    y3 = jnp.maximum(jnp.dot(y2, d3T_ref[...],
                             preferred_element_type=jnp.float32)
                     + b3d_ref[...], 0.0)
    xo = jnp.dot(y3, d4T_ref[...],
                 preferred_element_type=jnp.float32) + b4d_ref[...]
    xo = 2.0 * jax.nn.sigmoid(xo) + 0.2

    a_x = -(xo[:, 0:1] * s40 + xo[:, 1:2] * s42)
    a_y = -(xo[:, 2:3] * s41 + xo[:, 3:4] * s43)
    o_ref[...] = jnp.concatenate([a_x, a_y], axis=1)


def kernel(states, goals, conv_w1, conv_b1, conv_w2, conv_b2,
           dec_w1, dec_b1, dec_w2, dec_b2, dec_w3, dec_b3, dec_w4, dec_b4):
    n = states.shape[0]
    blk = 256 if n % 256 == 0 else n
    grid = n // blk

    statesT = states.T
    w1T = conv_w1.T
    w2T = conv_w2.T
    d1T = dec_w1.T
    d2T = dec_w2.T
    d3T = dec_w3.T
    d4T = dec_w4.T

    def full(a):
        return pl.BlockSpec(a.shape, lambda i: (0,) * a.ndim)

    b1 = conv_b1[None, :]
    b2 = conv_b2[None, :]
    b1d = dec_b1[None, :]
    b2d = dec_b2[None, :]
    b3d = dec_b3[None, :]
    b4d = dec_b4[None, :]

    return pl.pallas_call(
        functools.partial(_controller_body, blk=blk, n=n),
        grid=(grid,),
        in_specs=[
            pl.BlockSpec((blk, 4), lambda i: (i, 0)),
            full(states),
            full(statesT),
            pl.BlockSpec((blk, 2), lambda i: (i, 0)),
            full(w1T), full(b1), full(w2T), full(b2),
            full(d1T), full(b1d), full(d2T), full(b2d),
            full(d3T), full(b3d), full(d4T), full(b4d),
        ],
        out_specs=pl.BlockSpec((blk, 2), lambda i: (i, 0)),
        out_shape=jax.ShapeDtypeStruct((n, 2), jnp.float32),
    )(states, states, statesT, goals, w1T, b1, w2T, b2,
      d1T, b1d, d2T, b2d, d3T, b3d, d4T, b4d)


# speedup vs baseline: 3.0153x; 1.0004x over previous
import functools

import jax
import jax.numpy as jnp
from jax.experimental import pallas as pl

_TOPK = 12
_OBS_RADIUS = 1.0
_HIGH = jax.lax.Precision.HIGH


def _controller_body(s_ref, sfull_ref, sT_ref, g_ref, w1T_ref, b1_ref,
                     w2T_ref, b2_ref, d1T_ref, b1d_ref, d2T_ref,
                     b2d_ref, d3T_ref, b3d_ref, d4T_ref, b4d_ref, o_ref,
                     *, blk, n):
    i = pl.program_id(0)
    si = s_ref[...]
    xi = si[:, 0:1]
    yi = si[:, 1:2]
    xj = sT_ref[0:1, :]
    yj = sT_ref[1:2, :]

    dxm = xi - xj
    dym = yi - yj
    work = jnp.sqrt((dxm * dxm + 1e-4) + (dym * dym + 1e-4))

    iota = jax.lax.broadcasted_iota(jnp.int32, (blk, n), 1)
    gid = i * blk + jax.lax.broadcasted_iota(jnp.int32, (blk, 1), 0)

    sfull = sfull_ref[...]
    w1T = w1T_ref[...]
    b1 = b1_ref[...]
    w2T = w2T_ref[...]
    b2 = b2_ref[...]

    cols5 = []
    masks = []
    for k in range(_TOPK):
        m = jnp.min(work, axis=1, keepdims=True)
        idxs = jnp.min(jnp.where(work == m, iota, jnp.int32(2147483647)),
                       axis=1, keepdims=True)
        ohb = iota == idxs
        work = jnp.where(ohb, jnp.float32(1e30), work)
        ohf = ohb.astype(jnp.float32)
        gth = jnp.dot(ohf, sfull, precision=_HIGH,
                      preferred_element_type=jnp.float32)
        diff = si - gth
        eyef = (idxs == gid).astype(jnp.float32)
        dx = diff[:, 0:1]
        dy = diff[:, 1:2]
        dist = jnp.sqrt(dx * dx + dy * dy)
        masks.append((dist < _OBS_RADIUS).astype(jnp.float32))
        cols5.append(jnp.concatenate([diff, eyef], axis=1))

    best = None
    argf = None
    for kk in range(_TOPK):
        x5 = jnp.concatenate(
            [cols5[(c * _TOPK + kk) // 5]
             [:, (c * _TOPK + kk) % 5:(c * _TOPK + kk) % 5 + 1]
             for c in range(5)], axis=1)
        h1 = jnp.dot(x5, w1T,
                     preferred_element_type=jnp.float32) + b1
        h1 = jnp.maximum(h1, 0.0)
        h2 = jnp.dot(h1, w2T,
                     preferred_element_type=jnp.float32) + b2
        h2 = jnp.maximum(h2, 0.0)
        v = h2 * masks[kk]
        if kk == 0:
            best = v
            argf = jnp.zeros_like(v)
        else:
            upd = v > best
            argf = jnp.where(upd, jnp.float32(kk), argf)
            best = jnp.where(upd, v, best)

    gi = g_ref[...]
    s40 = xi - gi[:, 0:1]
    s41 = yi - gi[:, 1:2]
    s42 = si[:, 2:3]
    s43 = si[:, 3:4]

    xloc = jnp.concatenate([argf, s40, s41, s42, s43], axis=1)
    y1 = jnp.dot(xloc, d1T_ref[...],
                 preferred_element_type=jnp.float32) + b1d_ref[...]
    y1 = jnp.maximum(y1, 0.0)
    y2 = jnp.maximum(jnp.dot(y1, d2T_ref[...],
                             preferred_element_type=jnp.float32)
                     + b2d_ref[...], 0.0)
    y3 = jnp.maximum(jnp.dot(y2, d3T_ref[...],
                             preferred_element_type=jnp.float32)
                     + b3d_ref[...], 0.0)
    xo = jnp.dot(y3, d4T_ref[...],
                 preferred_element_type=jnp.float32) + b4d_ref[...]
    xo = 2.0 * jax.nn.sigmoid(xo) + 0.2

    a_x = -(xo[:, 0:1] * s40 + xo[:, 1:2] * s42)
    a_y = -(xo[:, 2:3] * s41 + xo[:, 3:4] * s43)
    o_ref[...] = jnp.concatenate([a_x, a_y], axis=1)


def kernel(states, goals, conv_w1, conv_b1, conv_w2, conv_b2,
           dec_w1, dec_b1, dec_w2, dec_b2, dec_w3, dec_b3, dec_w4, dec_b4):
    n = states.shape[0]
    blk = 256 if n % 256 == 0 else n
    grid = n // blk

    statesT = states.T
    w1T = conv_w1.T
    w2T = conv_w2.T
    d1T = dec_w1.T
    d2T = dec_w2.T
    d3T = dec_w3.T
    d4T = dec_w4.T

    def full(a):
        return pl.BlockSpec(a.shape, lambda i: (0,) * a.ndim)

    b1 = conv_b1[None, :]
    b2 = conv_b2[None, :]
    b1d = dec_b1[None, :]
    b2d = dec_b2[None, :]
    b3d = dec_b3[None, :]
    b4d = dec_b4[None, :]

    return pl.pallas_call(
        functools.partial(_controller_body, blk=blk, n=n),
        grid=(grid,),
        in_specs=[
            pl.BlockSpec((blk, 4), lambda i: (i, 0)),
            full(states),
            full(statesT),
            pl.BlockSpec((blk, 2), lambda i: (i, 0)),
            full(w1T), full(b1), full(w2T), full(b2),
            full(d1T), full(b1d), full(d2T), full(b2d),
            full(d3T), full(b3d), full(d4T), full(b4d),
        ],
        out_specs=pl.BlockSpec((blk, 2), lambda i: (i, 0)),
        out_shape=jax.ShapeDtypeStruct((n, 2), jnp.float32),
    )(states, states, statesT, goals, w1T, b1, w2T, b2,
      d1T, b1d, d2T, b2d, d3T, b3d, d4T, b4d)


# speedup vs baseline: 9.0070x; 2.9871x over previous
import functools

import jax
import jax.numpy as jnp
from jax.experimental import pallas as pl

_TOPK = 12
_OBS_RADIUS = 1.0
def _controller_body(s_ref, sfull_ref, sT_ref, g_ref, w1T_ref, b1_ref,
                     w2T_ref, b2_ref, d1T_ref, b1d_ref, d2T_ref,
                     b2d_ref, d3T_ref, b3d_ref, d4T_ref, b4d_ref, o_ref,
                     *, blk, n):
    i = pl.program_id(0)
    si = s_ref[...]
    xi = si[:, 0:1]
    yi = si[:, 1:2]
    xj = sT_ref[0:1, :]
    yj = sT_ref[1:2, :]

    dxm = xi - xj
    dym = yi - yj
    work = jnp.sqrt((dxm * dxm + 1e-4) + (dym * dym + 1e-4))

    iota = jax.lax.broadcasted_iota(jnp.int32, (blk, n), 1)
    gid = i * blk + jax.lax.broadcasted_iota(jnp.int32, (blk, 1), 0)

    sfull = sfull_ref[...]
    w1T = w1T_ref[...]
    b1 = b1_ref[...]
    w2T = w2T_ref[...]
    b2 = b2_ref[...]

    cols5 = []
    masks = []
    for k in range(_TOPK):
        m = jnp.min(work, axis=1, keepdims=True)
        idxs = jnp.min(jnp.where(work == m, iota, jnp.int32(2147483647)),
                       axis=1, keepdims=True)
        ohb = iota == idxs
        work = jnp.where(ohb, jnp.float32(1e30), work)
        ohf = ohb.astype(jnp.bfloat16)
        g3 = jnp.dot(ohf, sfull, preferred_element_type=jnp.float32)
        gth = (g3[:, 0:4] + g3[:, 4:8]) + g3[:, 8:12]
        diff = si - gth
        eyef = (idxs == gid).astype(jnp.float32)
        dx = diff[:, 0:1]
        dy = diff[:, 1:2]
        dist = jnp.sqrt(dx * dx + dy * dy)
        masks.append((dist < _OBS_RADIUS).astype(jnp.float32))
        cols5.append(jnp.concatenate([diff, eyef], axis=1))

    best = None
    argf = None
    for kk in range(_TOPK):
        x5 = jnp.concatenate(
            [cols5[(c * _TOPK + kk) // 5]
             [:, (c * _TOPK + kk) % 5:(c * _TOPK + kk) % 5 + 1]
             for c in range(5)], axis=1)
        h1 = jnp.dot(x5, w1T,
                     preferred_element_type=jnp.float32) + b1
        h1 = jnp.maximum(h1, 0.0)
        h2 = jnp.dot(h1, w2T,
                     preferred_element_type=jnp.float32) + b2
        h2 = jnp.maximum(h2, 0.0)
        v = h2 * masks[kk]
        if kk == 0:
            best = v
            argf = jnp.zeros_like(v)
        else:
            upd = v > best
            argf = jnp.where(upd, jnp.float32(kk), argf)
            best = jnp.where(upd, v, best)

    gi = g_ref[...]
    s40 = xi - gi[:, 0:1]
    s41 = yi - gi[:, 1:2]
    s42 = si[:, 2:3]
    s43 = si[:, 3:4]

    xloc = jnp.concatenate([argf, s40, s41, s42, s43], axis=1)
    y1 = jnp.dot(xloc, d1T_ref[...],
                 preferred_element_type=jnp.float32) + b1d_ref[...]
    y1 = jnp.maximum(y1, 0.0)
    y2 = jnp.maximum(jnp.dot(y1, d2T_ref[...],
                             preferred_element_type=jnp.float32)
                     + b2d_ref[...], 0.0)
    y3 = jnp.maximum(jnp.dot(y2, d3T_ref[...],
                             preferred_element_type=jnp.float32)
                     + b3d_ref[...], 0.0)
    xo = jnp.dot(y3, d4T_ref[...],
                 preferred_element_type=jnp.float32) + b4d_ref[...]
    xo = 2.0 * jax.nn.sigmoid(xo) + 0.2

    a_x = -(xo[:, 0:1] * s40 + xo[:, 1:2] * s42)
    a_y = -(xo[:, 2:3] * s41 + xo[:, 3:4] * s43)
    o_ref[...] = jnp.concatenate([a_x, a_y], axis=1)


def kernel(states, goals, conv_w1, conv_b1, conv_w2, conv_b2,
           dec_w1, dec_b1, dec_w2, dec_b2, dec_w3, dec_b3, dec_w4, dec_b4):
    n = states.shape[0]
    blk = 256 if n % 256 == 0 else n
    grid = n // blk

    statesT = states.T
    sh = states.astype(jnp.bfloat16)
    r1 = states - sh.astype(jnp.float32)
    sm = r1.astype(jnp.bfloat16)
    r2 = r1 - sm.astype(jnp.float32)
    sl = r2.astype(jnp.bfloat16)
    ssplit = jnp.concatenate([sh, sm, sl], axis=1)
    w1T = conv_w1.T
    w2T = conv_w2.T
    d1T = dec_w1.T
    d2T = dec_w2.T
    d3T = dec_w3.T
    d4T = dec_w4.T

    def full(a):
        return pl.BlockSpec(a.shape, lambda i: (0,) * a.ndim)

    b1 = conv_b1[None, :]
    b2 = conv_b2[None, :]
    b1d = dec_b1[None, :]
    b2d = dec_b2[None, :]
    b3d = dec_b3[None, :]
    b4d = dec_b4[None, :]

    return pl.pallas_call(
        functools.partial(_controller_body, blk=blk, n=n),
        grid=(grid,),
        in_specs=[
            pl.BlockSpec((blk, 4), lambda i: (i, 0)),
            full(ssplit),
            full(statesT),
            pl.BlockSpec((blk, 2), lambda i: (i, 0)),
            full(w1T), full(b1), full(w2T), full(b2),
            full(d1T), full(b1d), full(d2T), full(b2d),
            full(d3T), full(b3d), full(d4T), full(b4d),
        ],
        out_specs=pl.BlockSpec((blk, 2), lambda i: (i, 0)),
        out_shape=jax.ShapeDtypeStruct((n, 2), jnp.float32),
    )(states, ssplit, statesT, goals, w1T, b1, w2T, b2,
      d1T, b1d, d2T, b2d, d3T, b3d, d4T, b4d)


# speedup vs baseline: 9.4172x; 1.0455x over previous
import functools

import jax
import jax.numpy as jnp
from jax.experimental import pallas as pl
from jax.experimental.pallas import tpu as pltpu

_TOPK = 12
_OBS_RADIUS = 1.0
def _controller_body(s_ref, sfull_ref, sT_ref, g_ref, w1T_ref, b1_ref,
                     w2T_ref, b2_ref, d1T_ref, b1d_ref, d2T_ref,
                     b2d_ref, d3T_ref, b3d_ref, d4T_ref, b4d_ref, o_ref,
                     c5_ref, *, blk, n):
    i = pl.program_id(0)
    si = s_ref[...]
    xi = si[:, 0:1]
    yi = si[:, 1:2]
    xj = sT_ref[0:1, :]
    yj = sT_ref[1:2, :]

    dxm = xi - xj
    dym = yi - yj
    work = jnp.sqrt((dxm * dxm + 1e-4) + (dym * dym + 1e-4))

    iota = jax.lax.broadcasted_iota(jnp.int32, (blk, n), 1)
    gid = i * blk + jax.lax.broadcasted_iota(jnp.int32, (blk, 1), 0)

    sfull = sfull_ref[...]
    w1T = w1T_ref[...]
    b1 = b1_ref[...]
    w2T = w2T_ref[...]
    b2 = b2_ref[...]

    for k in range(_TOPK):
        m = jnp.min(work, axis=1, keepdims=True)
        idxs = jnp.min(jnp.where(work == m, iota, jnp.int32(2147483647)),
                       axis=1, keepdims=True)
        ohb = iota == idxs
        work = jnp.where(ohb, jnp.float32(1e30), work)
        ohf = ohb.astype(jnp.float32)
        g3 = jnp.dot(ohf, sfull, preferred_element_type=jnp.float32)
        gth = (g3[:, 0:4] + g3[:, 4:8]) + g3[:, 8:12]
        eyef = (idxs == gid).astype(jnp.float32)
        c5_ref[:, 5 * k:5 * k + 4] = si - gth
        c5_ref[:, 5 * k + 4:5 * k + 5] = eyef

    best = None
    argf = None
    for kk in range(_TOPK):
        x5 = jnp.concatenate(
            [c5_ref[:, c * _TOPK + kk:c * _TOPK + kk + 1]
             for c in range(5)], axis=1)
        dxk = c5_ref[:, 5 * kk:5 * kk + 1]
        dyk = c5_ref[:, 5 * kk + 1:5 * kk + 2]
        dist = jnp.sqrt(dxk * dxk + dyk * dyk)
        mk = (dist < _OBS_RADIUS).astype(jnp.float32)
        h1 = jnp.dot(x5, w1T,
                     preferred_element_type=jnp.float32) + b1
        h1 = jnp.maximum(h1, 0.0)
        h2 = jnp.dot(h1, w2T,
                     preferred_element_type=jnp.float32) + b2
        h2 = jnp.maximum(h2, 0.0)
        v = h2 * mk
        if kk == 0:
            best = v
            argf = jnp.zeros_like(v)
        else:
            upd = v > best
            argf = jnp.where(upd, jnp.float32(kk), argf)
            best = jnp.where(upd, v, best)

    gi = g_ref[...]
    s40 = xi - gi[:, 0:1]
    s41 = yi - gi[:, 1:2]
    s42 = si[:, 2:3]
    s43 = si[:, 3:4]

    xloc = jnp.concatenate([argf, s40, s41, s42, s43], axis=1)
    y1 = jnp.dot(xloc, d1T_ref[...],
                 preferred_element_type=jnp.float32) + b1d_ref[...]
    y1 = jnp.maximum(y1, 0.0)
    y2 = jnp.maximum(jnp.dot(y1, d2T_ref[...],
                             preferred_element_type=jnp.float32)
                     + b2d_ref[...], 0.0)
    y3 = jnp.maximum(jnp.dot(y2, d3T_ref[...],
                             preferred_element_type=jnp.float32)
                     + b3d_ref[...], 0.0)
    xo = jnp.dot(y3, d4T_ref[...],
                 preferred_element_type=jnp.float32) + b4d_ref[...]
    xo = 2.0 * jax.nn.sigmoid(xo) + 0.2

    a_x = -(xo[:, 0:1] * s40 + xo[:, 1:2] * s42)
    a_y = -(xo[:, 2:3] * s41 + xo[:, 3:4] * s43)
    o_ref[...] = jnp.concatenate([a_x, a_y], axis=1)


def kernel(states, goals, conv_w1, conv_b1, conv_w2, conv_b2,
           dec_w1, dec_b1, dec_w2, dec_b2, dec_w3, dec_b3, dec_w4, dec_b4):
    n = states.shape[0]
    blk = 256 if n % 256 == 0 else n
    grid = n // blk

    statesT = states.T
    sh = states.astype(jnp.bfloat16)
    r1 = states - sh.astype(jnp.float32)
    sm = r1.astype(jnp.bfloat16)
    r2 = r1 - sm.astype(jnp.float32)
    sl = r2.astype(jnp.bfloat16)
    ssplit = jnp.concatenate([sh.astype(jnp.float32),
                              sm.astype(jnp.float32),
                              sl.astype(jnp.float32)], axis=1)
    w1T = conv_w1.T
    w2T = conv_w2.T
    d1T = dec_w1.T
    d2T = dec_w2.T
    d3T = dec_w3.T
    d4T = dec_w4.T

    def full(a):
        return pl.BlockSpec(a.shape, lambda i: (0,) * a.ndim)

    b1 = conv_b1[None, :]
    b2 = conv_b2[None, :]
    b1d = dec_b1[None, :]
    b2d = dec_b2[None, :]
    b3d = dec_b3[None, :]
    b4d = dec_b4[None, :]

    return pl.pallas_call(
        functools.partial(_controller_body, blk=blk, n=n),
        grid=(grid,),
        in_specs=[
            pl.BlockSpec((blk, 4), lambda i: (i, 0)),
            full(ssplit),
            full(statesT),
            pl.BlockSpec((blk, 2), lambda i: (i, 0)),
            full(w1T), full(b1), full(w2T), full(b2),
            full(d1T), full(b1d), full(d2T), full(b2d),
            full(d3T), full(b3d), full(d4T), full(b4d),
        ],
        out_specs=pl.BlockSpec((blk, 2), lambda i: (i, 0)),
        out_shape=jax.ShapeDtypeStruct((n, 2), jnp.float32),
        scratch_shapes=[pltpu.VMEM((blk, 5 * _TOPK), jnp.float32)],
    )(states, ssplit, statesT, goals, w1T, b1, w2T, b2,
      d1T, b1d, d2T, b2d, d3T, b3d, d4T, b4d)
